# baseline (speedup 1.0000x reference)
import functools

import jax
import jax.numpy as jnp
from jax import lax
from jax.experimental import pallas as pl
from jax.experimental.pallas import tpu as pltpu
from jax.experimental.pallas import tpu_sc as plsc

NC = 2
NS = 16
NW = NC * NS
CHUNK = 128


def _body(table_hbm, idx_hbm, out_hbm, idx_v, rows_v, sem):
    nch = idx_hbm.shape[1]
    wid = lax.axis_index("s") * NC + lax.axis_index("c")
    pltpu.sync_copy(idx_hbm.at[wid], idx_v)

    def step(j, carry):
        pltpu.async_copy(table_hbm.at[idx_v.at[j]], rows_v, sem).wait()
        pltpu.sync_copy(rows_v, out_hbm.at[wid * nch + j])
        return carry

    lax.fori_loop(0, nch, step, 0)


@functools.lru_cache(maxsize=None)
def _make(nch, d):
    mesh = plsc.VectorSubcoreMesh(core_axis_name="c", subcore_axis_name="s")
    return pl.kernel(
        _body,
        out_type=jax.ShapeDtypeStruct((NW * nch, CHUNK, d), jnp.float32),
        mesh=mesh,
        scratch_types=[
            pltpu.VMEM((nch, CHUNK), jnp.int32),
            pltpu.VMEM((CHUNK, d), jnp.float32),
            pltpu.SemaphoreType.DMA,
        ],
        compiler_params=pltpu.CompilerParams(use_tc_tiling_on_sc=False),
    )


def kernel(word_input, table):
    b, s = word_input.shape
    vocab, d = table.shape
    idx = word_input.reshape(-1).astype(jnp.int32)
    total = b * s
    per_w = total // NW
    nch = per_w // CHUNK
    idx3 = idx.reshape(NW, nch, CHUNK)
    out = _make(nch, d)(table, idx3)
    return out.reshape(b, s, d)


# speedup vs baseline: 1.4512x; 1.4512x over previous
import functools

import jax
import jax.numpy as jnp
from jax import lax
from jax.experimental import pallas as pl
from jax.experimental.pallas import tpu as pltpu
from jax.experimental.pallas import tpu_sc as plsc

NC = 2
NS = 16
NW = NC * NS
CHUNK = 128
DPAD = 384


def _body(table_hbm, idx_hbm, out_hbm, idx_v, rows_v, sem):
    nch = idx_hbm.shape[1]
    wid = lax.axis_index("s") * NC + lax.axis_index("c")
    pltpu.sync_copy(idx_hbm.at[wid], idx_v)

    def step(j, carry):
        slot = lax.rem(j, 2)
        pltpu.async_copy(table_hbm.at[idx_v.at[j]], rows_v.at[slot], sem).wait()
        pltpu.sync_copy(rows_v.at[slot], out_hbm.at[wid * nch + j])
        return carry

    lax.fori_loop(0, nch, step, 0)


@functools.lru_cache(maxsize=None)
def _make(nch):
    mesh = plsc.VectorSubcoreMesh(core_axis_name="c", subcore_axis_name="s")
    return pl.kernel(
        _body,
        out_type=jax.ShapeDtypeStruct((NW * nch, CHUNK, DPAD), jnp.float32),
        mesh=mesh,
        scratch_types=[
            pltpu.VMEM((nch, CHUNK), jnp.int32),
            pltpu.VMEM((2, CHUNK, DPAD), jnp.float32),
            pltpu.SemaphoreType.DMA,
        ],
    )


def kernel(word_input, table):
    b, s = word_input.shape
    vocab, d = table.shape
    idx = word_input.reshape(-1).astype(jnp.int32)
    total = b * s
    per_w = total // NW
    nch = per_w // CHUNK
    idx3 = idx.reshape(NW, nch, CHUNK)
    tab_pad = jnp.pad(table, ((0, 0), (0, DPAD - d)))
    out = _make(nch)(tab_pad, idx3)
    return out[:, :, :d].reshape(b, s, d)
